# baseline (speedup 1.0000x reference)
import jax
import jax.numpy as jnp
from jax import lax
from jax.experimental import pallas as pl
from jax.experimental.pallas import tpu as pltpu
from jax.experimental.pallas import tpu_sc as plsc

N = 10000
E = 160000
F = 256
FH = 128
NT = 16
EPT = 10240
EP = NT * EPT
CH = 128
NCH = EPT // CH
ROWS_W = EP // 128 // 32
NP = 10240
STRIPE = NP // NT
RB = 1000


def _sc_mesh():
    return plsc.VectorSubcoreMesh(core_axis_name="c", subcore_axis_name="s")


def _deg_body(col_hbm, ew_hbm, out_hbm, col_v, ew_v, acc_v):
    c = lax.axis_index("c")
    s = lax.axis_index("s")
    w = s * 2 + c
    zf = jnp.zeros((16,), jnp.float32)

    def zero_body(i, _):
        acc_v[pl.ds(i * 16, 16)] = zf
        return 0

    lax.fori_loop(0, N // 16, zero_body, 0)
    pltpu.sync_copy(col_hbm.at[pl.ds(w * ROWS_W, ROWS_W)], col_v)
    pltpu.sync_copy(ew_hbm.at[pl.ds(w * ROWS_W, ROWS_W)], ew_v)

    def row_body(r, _):
        for g in range(8):
            idx = col_v[r, pl.ds(g * 16, 16)]
            val = ew_v[r, pl.ds(g * 16, 16)]
            plsc.addupdate_scatter(acc_v, [idx], val)
        return 0

    lax.fori_loop(0, ROWS_W, row_body, 0)
    pltpu.sync_copy(acc_v, out_hbm.at[w])


def _deg_partials(col2d, ew2d):
    k = pl.kernel(
        _deg_body,
        out_type=jax.ShapeDtypeStruct((32, N), jnp.float32),
        mesh=_sc_mesh(),
        compiler_params=pltpu.CompilerParams(needs_layout_passes=False),
        scratch_types=[
            pltpu.VMEM((ROWS_W, 128), jnp.int32),
            pltpu.VMEM((ROWS_W, 128), jnp.float32),
            pltpu.VMEM((N,), jnp.float32),
        ],
    )
    return k(col2d, ew2d)


def _agg_body(hs_hbm, row_hbm, cew_hbm, out_hbm,
              acc_sh, row_v, cbuf, gbuf,
              sem0, sem1, csem0, csem1, ssem0, ssem1):
    c = lax.axis_index("c")
    s = lax.axis_index("s")
    zf = jnp.zeros((16,), jnp.float32)
    base = s * STRIPE
    sems = (sem0, sem1)
    csems = (csem0, csem1)
    ssems = (ssem0, ssem1)

    def zero_body(i, _):
        for f in range(FH // 16):
            gbuf[0, i, pl.ds(f * 16, 16)] = zf
        return 0

    lax.fori_loop(0, CH, zero_body, 0)
    for t in range(STRIPE // CH):
        pltpu.sync_copy(gbuf.at[0], acc_sh.at[pl.ds(base + t * CH, CH)])

    pltpu.sync_copy(row_hbm.at[c, s], row_v)
    plsc.subcore_barrier()

    def start_fetch(k, b):
        pltpu.async_copy(hs_hbm.at[row_v.at[pl.ds(k * CH, CH)]],
                         gbuf.at[b], sems[b])
        pltpu.async_copy(cew_hbm.at[s, k], cbuf.at[b], csems[b])

    start_fetch(0, 0)
    start_fetch(1, 1)

    def chunk_body(j, _):
        for b in range(2):
            k = j * 2 + b
            pltpu.make_async_copy(hs_hbm.at[pl.ds(0, CH)], gbuf.at[b],
                                  sems[b]).wait()
            pltpu.make_async_copy(cew_hbm.at[0, 0], cbuf.at[b],
                                  csems[b]).wait()

            def scale_half(h, _):
                for ee in range(CH // 2):
                    e = h * (CH // 2) + ee
                    w16i = plsc.load_gather(
                        cbuf, [jnp.full((16,), b, jnp.int32),
                               jnp.full((16,), 1, jnp.int32),
                               jnp.full((16,), e, jnp.int32)])
                    w16 = plsc.bitcast(w16i, jnp.float32)
                    for f in range(FH // 16):
                        gbuf[b, e, pl.ds(f * 16, 16)] = (
                            gbuf[b, e, pl.ds(f * 16, 16)] * w16)
                return 0

            lax.fori_loop(0, 2, scale_half, 0)
            for g in range(CH // 16):
                cidx = cbuf[b, 0, pl.ds(g * 16, 16)]
                pltpu.async_copy(gbuf.at[b, pl.ds(g * 16, 16)],
                                 acc_sh.at[cidx], ssems[b], add=True)

        for b in range(2):
            k = j * 2 + b
            pltpu.make_async_copy(hs_hbm.at[pl.ds(0, CH)], gbuf.at[b],
                                  ssems[b]).wait()

            @pl.when(k + 2 < NCH)
            def _():
                start_fetch(k + 2, b)

        return 0

    lax.fori_loop(0, NCH // 2, chunk_body, 0)
    plsc.subcore_barrier()
    pltpu.sync_copy(acc_sh.at[pl.ds(base, STRIPE)],
                    out_hbm.at[pl.ds(c * NP + base, STRIPE)])


def _agg(hsb, rowb, cew):
    k = pl.kernel(
        _agg_body,
        out_type=jax.ShapeDtypeStruct((2 * NP, FH), jnp.float32),
        mesh=_sc_mesh(),
        compiler_params=pltpu.CompilerParams(needs_layout_passes=False),
        scratch_types=[
            pltpu.VMEM_SHARED((NP, FH), jnp.float32),
            pltpu.VMEM((EPT,), jnp.int32),
            pltpu.VMEM((2, 2, CH), jnp.int32),
            pltpu.VMEM((2, CH, FH), jnp.float32),
            pltpu.SemaphoreType.DMA,
            pltpu.SemaphoreType.DMA,
            pltpu.SemaphoreType.DMA,
            pltpu.SemaphoreType.DMA,
            pltpu.SemaphoreType.DMA,
            pltpu.SemaphoreType.DMA,
        ],
    )
    return k(hsb, rowb, cew)


def _interleave_bf16(hs):
    rb = hs.shape[0]
    g4 = hs.reshape(rb, F // 32, 2, 16)
    return jnp.swapaxes(g4, 2, 3).reshape(rb, F).astype(jnp.bfloat16)


def _tc1_body(parts_ref, x_ref, w1_ref, dinv_ref, hs_ref, hsb_ref):
    deg = jnp.sum(parts_ref[...], axis=1) + 1.0
    safe = jnp.where(deg > 0, deg, 1.0)
    dinv = jnp.where(deg > 0, lax.rsqrt(safe), 0.0)
    dinv_ref[:, 0] = dinv
    h = jnp.dot(x_ref[...], w1_ref[...], preferred_element_type=jnp.float32)
    hs = h * dinv[:, None]
    hs_ref[0] = hs[:, :FH]
    hs_ref[1] = hs[:, FH:]
    hsb = _interleave_bf16(hs)
    hsb_ref[0] = hsb[:, :FH]
    hsb_ref[1] = hsb[:, FH:]


def _tc1(parts_t, x, W1):
    return pl.pallas_call(
        _tc1_body,
        grid=(N // RB,),
        in_specs=[
            pl.BlockSpec((RB, 32), lambda i: (i, 0)),
            pl.BlockSpec((RB, F), lambda i: (i, 0)),
            pl.BlockSpec((F, F), lambda i: (0, 0)),
        ],
        out_specs=[
            pl.BlockSpec((RB, 1), lambda i: (i, 0)),
            pl.BlockSpec((2, RB, FH), lambda i: (0, i, 0)),
            pl.BlockSpec((2, RB, FH), lambda i: (0, i, 0)),
        ],
        out_shape=[
            jax.ShapeDtypeStruct((N, 1), jnp.float32),
            jax.ShapeDtypeStruct((2, N, FH), jnp.float32),
            jax.ShapeDtypeStruct((2, N, FH), jnp.bfloat16),
        ],
    )(parts_t, x, W1)


def _tc2_body(agg_ref, hs1_ref, dinv_ref, b1_ref, w2_ref, hs2_ref, hsb_ref):
    dinv = dinv_ref[:, 0]
    z = jnp.concatenate(
        [(agg_ref[q] + hs1_ref[q]) * dinv[:, None] for q in range(2)],
        axis=1) + b1_ref[0, :][None, :]
    z = jnp.maximum(z, 0.0)
    h2 = jnp.dot(z, w2_ref[...], preferred_element_type=jnp.float32)
    hs2 = h2 * dinv[:, None]
    hs2_ref[0] = hs2[:, :FH]
    hs2_ref[1] = hs2[:, FH:]
    hsb = _interleave_bf16(hs2)
    hsb_ref[0] = hsb[:, :FH]
    hsb_ref[1] = hsb[:, FH:]


def _tc2(agg1, hs1, dinv2d, b1, W2):
    return pl.pallas_call(
        _tc2_body,
        grid=(N // RB,),
        in_specs=[
            pl.BlockSpec((2, RB, FH), lambda i: (0, i, 0)),
            pl.BlockSpec((2, RB, FH), lambda i: (0, i, 0)),
            pl.BlockSpec((RB, 1), lambda i: (i, 0)),
            pl.BlockSpec((1, F), lambda i: (0, 0)),
            pl.BlockSpec((F, F), lambda i: (0, 0)),
        ],
        out_specs=[
            pl.BlockSpec((2, RB, FH), lambda i: (0, i, 0)),
            pl.BlockSpec((2, RB, FH), lambda i: (0, i, 0)),
        ],
        out_shape=[
            jax.ShapeDtypeStruct((2, N, FH), jnp.float32),
            jax.ShapeDtypeStruct((2, N, FH), jnp.bfloat16),
        ],
    )(agg1, hs1, dinv2d, b1, W2)


def _tc3_body(agg_ref, hs2_ref, dinv_ref, b2_ref, out_ref):
    dinv = dinv_ref[:, 0]
    o = jnp.concatenate(
        [(agg_ref[q] + hs2_ref[q]) * dinv[:, None] for q in range(2)],
        axis=1) + b2_ref[0, :][None, :]
    out_ref[...] = jnp.maximum(o, 0.0)


def _tc3(agg2, hs2, dinv2d, b2):
    return pl.pallas_call(
        _tc3_body,
        grid=(N // RB,),
        in_specs=[
            pl.BlockSpec((2, RB, FH), lambda i: (0, i, 0)),
            pl.BlockSpec((2, RB, FH), lambda i: (0, i, 0)),
            pl.BlockSpec((RB, 1), lambda i: (i, 0)),
            pl.BlockSpec((1, F), lambda i: (0, 0)),
        ],
        out_specs=pl.BlockSpec((RB, F), lambda i: (i, 0)),
        out_shape=jax.ShapeDtypeStruct((N, F), jnp.float32),
    )(agg2, hs2, dinv2d, b2)


def kernel(x, edge_index, edge_weight, W1, b1, W2, b2):
    row = edge_index[0].astype(jnp.int32)
    col = edge_index[1].astype(jnp.int32)
    ew = edge_weight.astype(jnp.float32)
    pad = EP - E
    rowp = jnp.concatenate([row, jnp.zeros((pad,), jnp.int32)])
    colp = jnp.concatenate([col, jnp.zeros((pad,), jnp.int32)])
    ewp = jnp.concatenate([ew, jnp.zeros((pad,), jnp.float32)])

    row2 = rowp.reshape(NT, EPT)
    rowb = jnp.stack([row2, row2 + N])
    cew = jnp.concatenate(
        [colp.reshape(NT, NCH, 1, CH),
         lax.bitcast_convert_type(ewp, jnp.int32).reshape(NT, NCH, 1, CH)],
        axis=2)
    col2d = colp.reshape(EP // 128, 128)
    ew2d = ewp.reshape(EP // 128, 128)

    parts = _deg_partials(col2d, ew2d)
    dinv2d, hs1, hsb1 = _tc1(parts.T, x, W1)

    agg1 = _agg(hs1.reshape(2 * N, FH), rowb, cew).reshape(2, NP, FH)
    hs2, hsb2 = _tc2(agg1, hs1, dinv2d, b1.reshape(1, F), W2)
    agg2 = _agg(hs2.reshape(2 * N, FH), rowb, cew).reshape(2, NP, FH)
    return _tc3(agg2, hs2, dinv2d, b2.reshape(1, F))


# speedup vs baseline: 1.9640x; 1.9640x over previous
import jax
import jax.numpy as jnp
from jax import lax
from jax.experimental import pallas as pl
from jax.experimental.pallas import tpu as pltpu
from jax.experimental.pallas import tpu_sc as plsc

N = 10000
E = 160000
F = 256
FH = 128
NT = 16
EPT = 10240
EP = NT * EPT
CH = 128
NCH = EPT // CH
ROWS_W = EP // 128 // 32
NP = 10240
STRIPE = NP // NT
RB = 1000


def _sc_mesh():
    return plsc.VectorSubcoreMesh(core_axis_name="c", subcore_axis_name="s")


def _deg_body(col_hbm, ew_hbm, out_hbm, col_v, ew_v, acc_v):
    c = lax.axis_index("c")
    s = lax.axis_index("s")
    w = s * 2 + c
    zf = jnp.zeros((16,), jnp.float32)

    def zero_body(i, _):
        acc_v[pl.ds(i * 16, 16)] = zf
        return 0

    lax.fori_loop(0, N // 16, zero_body, 0)
    pltpu.sync_copy(col_hbm.at[pl.ds(w * ROWS_W, ROWS_W)], col_v)
    pltpu.sync_copy(ew_hbm.at[pl.ds(w * ROWS_W, ROWS_W)], ew_v)

    def row_body(r, _):
        for g in range(8):
            idx = col_v[r, pl.ds(g * 16, 16)]
            val = ew_v[r, pl.ds(g * 16, 16)]
            plsc.addupdate_scatter(acc_v, [idx], val)
        return 0

    lax.fori_loop(0, ROWS_W, row_body, 0)
    pltpu.sync_copy(acc_v, out_hbm.at[w])


def _deg_partials(col2d, ew2d):
    k = pl.kernel(
        _deg_body,
        out_type=jax.ShapeDtypeStruct((32, N), jnp.float32),
        mesh=_sc_mesh(),
        compiler_params=pltpu.CompilerParams(needs_layout_passes=False),
        scratch_types=[
            pltpu.VMEM((ROWS_W, 128), jnp.int32),
            pltpu.VMEM((ROWS_W, 128), jnp.float32),
            pltpu.VMEM((N,), jnp.float32),
        ],
    )
    return k(col2d, ew2d)


def _agg_body(hs_hbm, row_hbm, cew_hbm, out_hbm,
              acc_sh, row_v, cbuf, gbuf, sem0, sem1, csem0, csem1):
    c = lax.axis_index("c")
    s = lax.axis_index("s")
    zf = jnp.zeros((16,), jnp.float32)
    base = s * STRIPE
    sems = (sem0, sem1)
    csems = (csem0, csem1)

    def zero_body(i, _):
        for f in range(FH // 16):
            gbuf[0, i, pl.ds(f * 16, 16)] = zf
        return 0

    lax.fori_loop(0, CH, zero_body, 0)
    for t in range(STRIPE // CH):
        pltpu.sync_copy(gbuf.at[0], acc_sh.at[pl.ds(base + t * CH, CH)])

    pltpu.sync_copy(row_hbm.at[c, s], row_v)
    plsc.subcore_barrier()

    def start_fetch(k, b):
        pltpu.async_copy(hs_hbm.at[row_v.at[pl.ds(k * CH, CH)]],
                         gbuf.at[b], sems[b])
        pltpu.async_copy(cew_hbm.at[s, k], cbuf.at[b], csems[b])

    start_fetch(0, 0)
    start_fetch(1, 1)

    def chunk_body(j, _):
        for b in range(2):
            k = j * 2 + b
            pltpu.make_async_copy(hs_hbm.at[pl.ds(0, CH)], gbuf.at[b],
                                  sems[b]).wait()
            pltpu.make_async_copy(cew_hbm.at[0, 0], cbuf.at[b],
                                  csems[b]).wait()

            def scale_half(h, _):
                for ee in range(CH // 2):
                    e = h * (CH // 2) + ee
                    w16i = plsc.load_gather(
                        cbuf, [jnp.full((16,), b, jnp.int32),
                               jnp.full((16,), 1, jnp.int32),
                               jnp.full((16,), e, jnp.int32)])
                    w16 = plsc.bitcast(w16i, jnp.float32)
                    for f in range(FH // 16):
                        gbuf[b, e, pl.ds(f * 16, 16)] = (
                            gbuf[b, e, pl.ds(f * 16, 16)] * w16)
                return 0

            lax.fori_loop(0, 2, scale_half, 0)
            for g in range(CH // 16):
                cidx = cbuf[b, 0, pl.ds(g * 16, 16)]
                pltpu.sync_copy(gbuf.at[b, pl.ds(g * 16, 16)],
                                acc_sh.at[cidx], add=True)

            @pl.when(k + 2 < NCH)
            def _():
                start_fetch(k + 2, b)

        return 0

    lax.fori_loop(0, NCH // 2, chunk_body, 0)
    plsc.subcore_barrier()
    pltpu.sync_copy(acc_sh.at[pl.ds(base, STRIPE)],
                    out_hbm.at[pl.ds(c * NP + base, STRIPE)])


def _agg(hsb, rowb, cew):
    k = pl.kernel(
        _agg_body,
        out_type=jax.ShapeDtypeStruct((2 * NP, FH), jnp.float32),
        mesh=_sc_mesh(),
        compiler_params=pltpu.CompilerParams(needs_layout_passes=False),
        scratch_types=[
            pltpu.VMEM_SHARED((NP, FH), jnp.float32),
            pltpu.VMEM((EPT,), jnp.int32),
            pltpu.VMEM((2, 2, CH), jnp.int32),
            pltpu.VMEM((2, CH, FH), jnp.float32),
            pltpu.SemaphoreType.DMA,
            pltpu.SemaphoreType.DMA,
            pltpu.SemaphoreType.DMA,
            pltpu.SemaphoreType.DMA,
        ],
    )
    return k(hsb, rowb, cew)


def _tc1_body(parts_ref, x_ref, w1_ref, dinv_ref, hs_ref):
    deg = jnp.sum(parts_ref[...], axis=1) + 1.0
    safe = jnp.where(deg > 0, deg, 1.0)
    dinv = jnp.where(deg > 0, lax.rsqrt(safe), 0.0)
    dinv_ref[:, 0] = dinv
    h = jnp.dot(x_ref[...], w1_ref[...], preferred_element_type=jnp.float32)
    hs = h * dinv[:, None]
    hs_ref[0] = hs[:, :FH]
    hs_ref[1] = hs[:, FH:]


def _tc1(parts_t, x, W1):
    return pl.pallas_call(
        _tc1_body,
        grid=(N // RB,),
        in_specs=[
            pl.BlockSpec((RB, 32), lambda i: (i, 0)),
            pl.BlockSpec((RB, F), lambda i: (i, 0)),
            pl.BlockSpec((F, F), lambda i: (0, 0)),
        ],
        out_specs=[
            pl.BlockSpec((RB, 1), lambda i: (i, 0)),
            pl.BlockSpec((2, RB, FH), lambda i: (0, i, 0)),
        ],
        out_shape=[
            jax.ShapeDtypeStruct((N, 1), jnp.float32),
            jax.ShapeDtypeStruct((2, N, FH), jnp.float32),
        ],
    )(parts_t, x, W1)


def _tc2_body(agg_ref, hs1_ref, dinv_ref, b1_ref, w2_ref, hs2_ref):
    dinv = dinv_ref[:, 0]
    z = jnp.concatenate(
        [(agg_ref[q] + hs1_ref[q]) * dinv[:, None] for q in range(2)],
        axis=1) + b1_ref[0, :][None, :]
    z = jnp.maximum(z, 0.0)
    h2 = jnp.dot(z, w2_ref[...], preferred_element_type=jnp.float32)
    hs2 = h2 * dinv[:, None]
    hs2_ref[0] = hs2[:, :FH]
    hs2_ref[1] = hs2[:, FH:]


def _tc2(agg1, hs1, dinv2d, b1, W2):
    return pl.pallas_call(
        _tc2_body,
        grid=(N // RB,),
        in_specs=[
            pl.BlockSpec((2, RB, FH), lambda i: (0, i, 0)),
            pl.BlockSpec((2, RB, FH), lambda i: (0, i, 0)),
            pl.BlockSpec((RB, 1), lambda i: (i, 0)),
            pl.BlockSpec((1, F), lambda i: (0, 0)),
            pl.BlockSpec((F, F), lambda i: (0, 0)),
        ],
        out_specs=pl.BlockSpec((2, RB, FH), lambda i: (0, i, 0)),
        out_shape=jax.ShapeDtypeStruct((2, N, FH), jnp.float32),
    )(agg1, hs1, dinv2d, b1, W2)


def _tc3_body(agg_ref, hs2_ref, dinv_ref, b2_ref, out_ref):
    dinv = dinv_ref[:, 0]
    o = jnp.concatenate(
        [(agg_ref[q] + hs2_ref[q]) * dinv[:, None] for q in range(2)],
        axis=1) + b2_ref[0, :][None, :]
    out_ref[...] = jnp.maximum(o, 0.0)


def _tc3(agg2, hs2, dinv2d, b2):
    return pl.pallas_call(
        _tc3_body,
        grid=(N // RB,),
        in_specs=[
            pl.BlockSpec((2, RB, FH), lambda i: (0, i, 0)),
            pl.BlockSpec((2, RB, FH), lambda i: (0, i, 0)),
            pl.BlockSpec((RB, 1), lambda i: (i, 0)),
            pl.BlockSpec((1, F), lambda i: (0, 0)),
        ],
        out_specs=pl.BlockSpec((RB, F), lambda i: (i, 0)),
        out_shape=jax.ShapeDtypeStruct((N, F), jnp.float32),
    )(agg2, hs2, dinv2d, b2)


def kernel(x, edge_index, edge_weight, W1, b1, W2, b2):
    row = edge_index[0].astype(jnp.int32)
    col = edge_index[1].astype(jnp.int32)
    ew = edge_weight.astype(jnp.float32)
    pad = EP - E
    rowp = jnp.concatenate([row, jnp.zeros((pad,), jnp.int32)])
    colp = jnp.concatenate([col, jnp.zeros((pad,), jnp.int32)])
    ewp = jnp.concatenate([ew, jnp.zeros((pad,), jnp.float32)])

    row2 = rowp.reshape(NT, EPT)
    rowb = jnp.stack([row2, row2 + N])
    cew = jnp.concatenate(
        [colp.reshape(NT, NCH, 1, CH),
         lax.bitcast_convert_type(ewp, jnp.int32).reshape(NT, NCH, 1, CH)],
        axis=2)
    col2d = colp.reshape(EP // 128, 128)
    ew2d = ewp.reshape(EP // 128, 128)

    parts = _deg_partials(col2d, ew2d)
    dinv2d, hs1 = _tc1(parts.T, x, W1)

    agg1 = _agg(hs1.reshape(2 * N, FH), rowb, cew).reshape(2, NP, FH)
    hs2 = _tc2(agg1, hs1, dinv2d, b1.reshape(1, F), W2)
    agg2 = _agg(hs2.reshape(2 * N, FH), rowb, cew).reshape(2, NP, FH)
    return _tc3(agg2, hs2, dinv2d, b2.reshape(1, F))
